# Initial kernel scaffold; baseline (speedup 1.0000x reference)
import jax
import jax.numpy as jnp
from jax.experimental import pallas as pl


def kernel(input_ids, attention_mask, emb_table, fc_w, fc_b):
    raise NotImplementedError("write your pallas kernel here")


# speedup vs baseline: 3.5494x; 3.5494x over previous
import functools

import jax
import jax.numpy as jnp
from jax import lax
from jax.experimental import pallas as pl
from jax.experimental.pallas import tpu as pltpu
from jax.experimental.pallas import tpu_sc as plsc

_VOCAB = 1000000
_D = 64
_B = 16384
_L = 200
_N = _B * _L

_VB = 25000
_NVB = _VOCAB // _VB

_NC = 2
_NS = 16
_NW = _NC * _NS
_PER_W = _N // _NW
_CH = 25600
_NCHUNK = _PER_W // _CH

_BB = 2048


def _project_body(emb_ref, w_ref, out_ref):
    x = emb_ref[...]
    w = w_ref[...]
    y = jnp.sum(x * w, axis=1)
    out_ref[...] = y.reshape(1, 1, _VB)


def _project_table(emb_table, fc_w):
    out = pl.pallas_call(
        _project_body,
        grid=(_NVB,),
        in_specs=[
            pl.BlockSpec((_VB, _D), lambda i: (i, 0)),
            pl.BlockSpec((1, _D), lambda i: (0, 0)),
        ],
        out_specs=pl.BlockSpec((1, 1, _VB), lambda i: (i, 0, 0)),
        out_shape=jax.ShapeDtypeStruct((_NVB, 1, _VB), jnp.float32),
    )(emb_table, fc_w)
    return out.reshape(_VOCAB)


def _sc_gather(p, idx_flat):
    mesh = plsc.VectorSubcoreMesh(core_axis_name="c", subcore_axis_name="s")

    @functools.partial(
        pl.kernel,
        out_type=jax.ShapeDtypeStruct((_N,), jnp.float32),
        mesh=mesh,
        scratch_types=[
            pltpu.VMEM((_CH,), jnp.int32),
            pltpu.VMEM((_CH,), jnp.float32),
            pltpu.SemaphoreType.DMA,
        ],
    )
    def gather_kernel(p_hbm, idx_hbm, out_hbm, idx_v, vals_v, sem):
        wid = lax.axis_index("s") * _NC + lax.axis_index("c")
        base = wid * _PER_W

        @pl.loop(0, _NCHUNK)
        def _chunk(k):
            off = base + k * _CH
            pltpu.sync_copy(idx_hbm.at[pl.ds(off, _CH)], idx_v)
            pltpu.async_copy(p_hbm.at[idx_v], vals_v, sem).wait()
            pltpu.sync_copy(vals_v, out_hbm.at[pl.ds(off, _CH)])

    return gather_kernel(p, idx_flat)


def _pool_body(vals_ref, mask_ref, b_ref, out_ref):
    m = mask_ref[...].astype(jnp.float32)
    v = vals_ref[...]
    s = jnp.sum(m, axis=1, keepdims=True)
    acc = jnp.sum(v * m, axis=1, keepdims=True)
    out_ref[...] = acc / s + b_ref[0, 0]


def _pool(vals2d, mask, fc_b):
    b2d = fc_b.reshape(1, 1)
    return pl.pallas_call(
        _pool_body,
        grid=(_B // _BB,),
        in_specs=[
            pl.BlockSpec((_BB, _L), lambda i: (i, 0)),
            pl.BlockSpec((_BB, _L), lambda i: (i, 0)),
            pl.BlockSpec((1, 1), lambda i: (0, 0)),
        ],
        out_specs=pl.BlockSpec((_BB, 1), lambda i: (i, 0)),
        out_shape=jax.ShapeDtypeStruct((_B, 1), jnp.float32),
    )(vals2d, mask, b2d)


def kernel(input_ids, attention_mask, emb_table, fc_w, fc_b):
    p = _project_table(emb_table, fc_w)
    idx_flat = input_ids.astype(jnp.int32).reshape(_N)
    vals = _sc_gather(p, idx_flat)
    vals2d = vals.reshape(_B, _L)
    return _pool(vals2d, attention_mask.astype(jnp.int32), fc_b)
